# baseline (speedup 1.0000x reference)
import dataclasses
import functools

import jax
import jax.numpy as jnp
from jax import lax
from jax.experimental import pallas as pl
from jax.experimental.pallas import tpu as pltpu
from jax.experimental.pallas import tpu_sc as plsc

A, N, K, F, E = 2, 10000, 16, 128, 4
NC, NS = 2, 16
NW = NC * NS
WIN = 8
NT = A * N
NPW = 640
NT_PAD = NW * NPW
NWIN = NPW // WIN

_HIGHEST = lax.Precision.DEFAULT


def _he_body(x_ref, w_ref, he_ref):
    x = x_ref[0]
    for e in range(E):
        he_ref[0, e] = lax.dot_general(
            x, w_ref[0, e], (((1,), (0,)), ((), ())),
            preferred_element_type=jnp.float32, precision=_HIGHEST)


def _he_call(x, pw_W, bn=2000):
    nb = N // bn
    return pl.pallas_call(
        _he_body,
        grid=(A, nb),
        in_specs=[
            pl.BlockSpec((1, bn, F), lambda a, i: (a, i, 0)),
            pl.BlockSpec((1, E, F, F), lambda a, i: (a, 0, 0, 0)),
        ],
        out_specs=pl.BlockSpec((1, E, bn, F), lambda a, i: (a, 0, i, 0)),
        out_shape=jax.ShapeDtypeStruct((A, E, N, F), jnp.float32),
    )(x, pw_W)


_mesh = plsc.VectorSubcoreMesh(core_axis_name="c", subcore_axis_name="s")

_sc_params = pltpu.CompilerParams()
if "needs_layout_passes" in pltpu.CompilerParams.__dataclass_fields__:
    _sc_params = dataclasses.replace(_sc_params, needs_layout_passes=False)


def _accum_window(g, ob, row0):
    @pl.loop(0, WIN)
    def _(cc):
        base = cc * K
        for grp in range(F // 16):
            sl = pl.ds(grp * 16, 16)
            acc = g[base, sl]
            for k in range(1, K):
                acc = acc + g[base + k, sl]
            ob[row0 + cc, sl] = acc * (1.0 / K)


@functools.partial(
    pl.kernel,
    out_type=jax.ShapeDtypeStruct((NT_PAD, F), jnp.float32),
    mesh=_mesh,
    scratch_types=[
        pltpu.VMEM((NWIN, WIN * K), jnp.int32),
        pltpu.VMEM((4, WIN * K, F), jnp.float32),
        pltpu.VMEM((4, WIN, F), jnp.float32),
        pltpu.SemaphoreType.DMA,
        pltpu.SemaphoreType.DMA,
        pltpu.SemaphoreType.DMA,
        pltpu.SemaphoreType.DMA,
        pltpu.SemaphoreType.DMA,
        pltpu.SemaphoreType.DMA,
        pltpu.SemaphoreType.DMA,
        pltpu.SemaphoreType.DMA,
        pltpu.SemaphoreType.DMA,
    ],
    compiler_params=_sc_params,
)
def _sc_gather_mean(he_hbm, idx_hbm, out_hbm, idx_v, gbuf, obuf,
                    sem_i, sg0, sg1, sg2, sg3, so0, so1, so2, so3):
    cid = lax.axis_index("c")
    sid = lax.axis_index("s")
    wid = cid * NS + sid
    sgs = (sg0, sg1, sg2, sg3)
    sos = (so0, so1, so2, so3)

    pltpu.async_copy(idx_hbm.at[wid], idx_v, sem_i).wait()
    for w in range(3):
        pltpu.async_copy(he_hbm.at[idx_v.at[w]], gbuf.at[w], sgs[w])

    @pl.loop(0, NWIN, step=4)
    def _(j):
        for b in range(4):
            jw = j + b
            pltpu.make_async_copy(he_hbm.at[idx_v.at[jw]], gbuf.at[b],
                                  sgs[b]).wait()
            nxt = jw + 3
            bn = (b + 3) % 4

            @pl.when(nxt < NWIN)
            def _():
                pltpu.async_copy(he_hbm.at[idx_v.at[nxt]], gbuf.at[bn],
                                 sgs[bn])

            @pl.when(jw >= 4)
            def _():
                pltpu.make_async_copy(obuf.at[b], out_hbm.at[pl.ds(0, WIN)],
                                      sos[b]).wait()

            _accum_window(gbuf.at[b], obuf.at[b], 0)
            ofs = pl.multiple_of((wid * NWIN + jw) * WIN, 8)
            pltpu.async_copy(obuf.at[b], out_hbm.at[pl.ds(ofs, WIN)], sos[b])

    for b in range(4):
        pltpu.make_async_copy(obuf.at[b], out_hbm.at[pl.ds(0, WIN)],
                              sos[b]).wait()


def _post_body(x_ref, agg0_ref, agg1_ref, sw_ref, pwb_ref, u_ref, v_ref,
               hb_ref, out_ref):
    dims = (((1,), (0,)), ((), ()))
    aggs = (agg0_ref, agg1_ref)
    pw = []
    for a in range(A):
        h = lax.dot_general(x_ref[a], sw_ref[a], dims,
                            preferred_element_type=jnp.float32,
                            precision=_HIGHEST)
        pw.append(jnp.maximum(aggs[a][...] + h + pwb_ref[a, 0], 0.0))
    meg = pw[0] + pw[1]
    for a in range(A):
        h = (lax.dot_general(pw[a], u_ref[a], dims,
                             preferred_element_type=jnp.float32,
                             precision=_HIGHEST)
             + lax.dot_general(meg, v_ref[a], dims,
                               preferred_element_type=jnp.float32,
                               precision=_HIGHEST)
             + hb_ref[a, 0])
        out_ref[a] = jnp.maximum(h, 0.0) + x_ref[a]


def _post_call(x, agg_pad, pw_selfW, pw_b, hop_U, hop_V, hop_b, bn=2000):
    nb = N // bn
    full = lambda i: (0, 0, 0)
    return pl.pallas_call(
        _post_body,
        grid=(nb,),
        in_specs=[
            pl.BlockSpec((A, bn, F), lambda i: (0, i, 0)),
            pl.BlockSpec((bn, F), lambda i: (i, 0)),
            pl.BlockSpec((bn, F), lambda i: (N // bn + i, 0)),
            pl.BlockSpec((A, F, F), full),
            pl.BlockSpec((A, 1, F), full),
            pl.BlockSpec((A, F, F), full),
            pl.BlockSpec((A, F, F), full),
            pl.BlockSpec((A, 1, F), full),
        ],
        out_specs=pl.BlockSpec((A, bn, F), lambda i: (0, i, 0)),
        out_shape=jax.ShapeDtypeStruct((A, N, F), jnp.float32),
    )(x, agg_pad, agg_pad, pw_selfW, pw_b.reshape(A, 1, F), hop_U, hop_V,
      hop_b.reshape(A, 1, F))


def kernel(nfeature, nn_idx, etype, pw_W, pw_selfW, pw_b, hop_U, hop_V, hop_b):
    x = nfeature[0]
    nn = nn_idx[0].astype(jnp.int32)
    et = etype[0].astype(jnp.int32)

    aofs = (jnp.arange(A, dtype=jnp.int32) * E)[:, None, None]
    fi = ((et + aofs) * N + nn).reshape(NT * K)
    pad = (jnp.arange((NT_PAD - NT) * K, dtype=jnp.int32) * 523) % (A * E * N)
    fi = jnp.concatenate([fi, pad]).reshape(NW, NWIN, WIN * K)

    he = _he_call(x, pw_W).reshape(A * E * N, F)
    agg_pad = _sc_gather_mean(he, fi)
    out = _post_call(x, agg_pad, pw_selfW, pw_b, hop_U, hop_V, hop_b)
    return out[None]


# speedup vs baseline: 1.0730x; 1.0730x over previous
import dataclasses
import functools

import jax
import jax.numpy as jnp
from jax import lax
from jax.experimental import pallas as pl
from jax.experimental.pallas import tpu as pltpu
from jax.experimental.pallas import tpu_sc as plsc

A, N, K, F, E = 2, 10000, 16, 128, 4
NC, NS = 2, 16
NW = NC * NS
WIN = 8
NT = A * N
NPW = 640
NT_PAD = NW * NPW
NWIN = NPW // WIN

_HIGHEST = lax.Precision.DEFAULT


def _he_body(x_ref, w_ref, he_ref):
    x = x_ref[0]
    for e in range(E):
        he_ref[0, e] = lax.dot_general(
            x, w_ref[0, e], (((1,), (0,)), ((), ())),
            preferred_element_type=jnp.float32, precision=_HIGHEST)


def _he_call(x, pw_W, bn=2000):
    nb = N // bn
    return pl.pallas_call(
        _he_body,
        grid=(A, nb),
        in_specs=[
            pl.BlockSpec((1, bn, F), lambda a, i: (a, i, 0)),
            pl.BlockSpec((1, E, F, F), lambda a, i: (a, 0, 0, 0)),
        ],
        out_specs=pl.BlockSpec((1, E, bn, F), lambda a, i: (a, 0, i, 0)),
        out_shape=jax.ShapeDtypeStruct((A, E, N, F), jnp.float32),
    )(x, pw_W)


_mesh = plsc.VectorSubcoreMesh(core_axis_name="c", subcore_axis_name="s")

_sc_params = pltpu.CompilerParams()
if "needs_layout_passes" in pltpu.CompilerParams.__dataclass_fields__:
    _sc_params = dataclasses.replace(_sc_params, needs_layout_passes=False)


def _accum_window(g, ob, row0):
    @pl.loop(0, WIN)
    def _(cc):
        base = cc * K
        for grp in range(F // 16):
            sl = pl.ds(grp * 16, 16)
            acc = g[base, sl]
            for k in range(1, K):
                acc = acc + g[base + k, sl]
            ob[row0 + cc, sl] = acc * (1.0 / K)


@functools.partial(
    pl.kernel,
    out_type=jax.ShapeDtypeStruct((NT_PAD, F), jnp.float32),
    mesh=_mesh,
    scratch_types=[
        pltpu.VMEM((NWIN, WIN * K), jnp.int32),
        pltpu.VMEM((WIN * K, F), jnp.float32),
        pltpu.VMEM((WIN * K, F), jnp.float32),
        pltpu.VMEM((NPW, F), jnp.float32),
        pltpu.SemaphoreType.DMA,
        pltpu.SemaphoreType.DMA,
        pltpu.SemaphoreType.DMA,
    ],
    compiler_params=_sc_params,
)
def _sc_gather_mean(he_hbm, idx_hbm, out_hbm, idx_v, ga, gb, obuf,
                    sem_i, sem_a, sem_b):
    cid = lax.axis_index("c")
    sid = lax.axis_index("s")
    wid = cid * NS + sid

    pltpu.async_copy(idx_hbm.at[wid], idx_v, sem_i).wait()
    pltpu.async_copy(he_hbm.at[idx_v.at[0]], ga, sem_a)

    @pl.loop(0, NWIN, step=2)
    def _(j):
        pltpu.async_copy(he_hbm.at[idx_v.at[j + 1]], gb, sem_b)
        pltpu.make_async_copy(he_hbm.at[idx_v.at[j]], ga, sem_a).wait()
        _accum_window(ga, obuf, j * WIN)

        @pl.when(j + 2 < NWIN)
        def _():
            pltpu.async_copy(he_hbm.at[idx_v.at[j + 2]], ga, sem_a)

        pltpu.make_async_copy(he_hbm.at[idx_v.at[j + 1]], gb, sem_b).wait()
        _accum_window(gb, obuf, (j + 1) * WIN)

    pltpu.sync_copy(obuf, out_hbm.at[pl.ds(wid * NPW, NPW)])


def _post_body(x_ref, agg0_ref, agg1_ref, sw_ref, pwb_ref, u_ref, v_ref,
               hb_ref, out_ref):
    dims = (((1,), (0,)), ((), ()))
    aggs = (agg0_ref, agg1_ref)
    pw = []
    for a in range(A):
        h = lax.dot_general(x_ref[a], sw_ref[a], dims,
                            preferred_element_type=jnp.float32,
                            precision=_HIGHEST)
        pw.append(jnp.maximum(aggs[a][...] + h + pwb_ref[a, 0], 0.0))
    meg = pw[0] + pw[1]
    for a in range(A):
        h = (lax.dot_general(pw[a], u_ref[a], dims,
                             preferred_element_type=jnp.float32,
                             precision=_HIGHEST)
             + lax.dot_general(meg, v_ref[a], dims,
                               preferred_element_type=jnp.float32,
                               precision=_HIGHEST)
             + hb_ref[a, 0])
        out_ref[a] = jnp.maximum(h, 0.0) + x_ref[a]


def _post_call(x, agg_pad, pw_selfW, pw_b, hop_U, hop_V, hop_b, bn=2000):
    nb = N // bn
    full = lambda i: (0, 0, 0)
    return pl.pallas_call(
        _post_body,
        grid=(nb,),
        in_specs=[
            pl.BlockSpec((A, bn, F), lambda i: (0, i, 0)),
            pl.BlockSpec((bn, F), lambda i: (i, 0)),
            pl.BlockSpec((bn, F), lambda i: (N // bn + i, 0)),
            pl.BlockSpec((A, F, F), full),
            pl.BlockSpec((A, 1, F), full),
            pl.BlockSpec((A, F, F), full),
            pl.BlockSpec((A, F, F), full),
            pl.BlockSpec((A, 1, F), full),
        ],
        out_specs=pl.BlockSpec((A, bn, F), lambda i: (0, i, 0)),
        out_shape=jax.ShapeDtypeStruct((A, N, F), jnp.float32),
    )(x, agg_pad, agg_pad, pw_selfW, pw_b.reshape(A, 1, F), hop_U, hop_V,
      hop_b.reshape(A, 1, F))


def kernel(nfeature, nn_idx, etype, pw_W, pw_selfW, pw_b, hop_U, hop_V, hop_b):
    x = nfeature[0]
    nn = nn_idx[0].astype(jnp.int32)
    et = etype[0].astype(jnp.int32)

    aofs = (jnp.arange(A, dtype=jnp.int32) * E)[:, None, None]
    fi = ((et + aofs) * N + nn).reshape(NT * K)
    pad = (jnp.arange((NT_PAD - NT) * K, dtype=jnp.int32) * 523) % (A * E * N)
    fi = jnp.concatenate([fi, pad]).reshape(NW, NWIN, WIN * K)

    he = _he_call(x, pw_W).reshape(A * E * N, F)
    agg_pad = _sc_gather_mean(he, fi)
    out = _post_call(x, agg_pad, pw_selfW, pw_b, hop_U, hop_V, hop_b)
    return out[None]
